# baseline (speedup 1.0000x reference)
import jax
import jax.numpy as jnp
from jax import lax
from jax.experimental import pallas as pl
from jax.experimental.pallas import tpu as pltpu
from jax.experimental.pallas import tpu_sc as plsc

N = 10000
E = 320000
D = 128
NLAYERS = 4
G = 16

NC, NS, LANES = 2, 16, 16
NW = NC * NS
EB = 128
NBLK = E // EB
RPW = (NBLK + NW - 1) // NW
ZCH = 640
NPAD = NS * ZCH

BN = 2000
BE = 2560
XW = 16
PW = 128
TW = D + PW

_f32 = jnp.float32
_DBG_XLA_SCATTER = False


def _silu(v):
    return v * jax.nn.sigmoid(v)


def _dot(a, b):
    return jnp.dot(a, b, preferred_element_type=_f32,
                   precision=jax.lax.Precision.HIGHEST)


def _sc_gather_body(dst_hbm, src_hbm, a1h, a2h, t1x, t2x,
                    preh_hbm, prex_hbm,
                    dstv, srcv, bufa, bufb, bufc, bufd,
                    sem0, sem1, sem2, sem3):
    cid = lax.axis_index("c")
    sid = lax.axis_index("s")
    wid = sid * NC + cid

    def addh_row(i, _):
        for j in range(D // LANES):
            b = bufb[i, pl.ds(j * LANES, LANES)]
            plsc.addupdate(bufa.at[i, pl.ds(j * LANES, LANES)], b)
        return 0

    def addx_row(i, _):
        for j in range(D // LANES):
            b = bufd[i, pl.ds(j * LANES, LANES)]
            plsc.addupdate(bufc.at[i, pl.ds(j * LANES, LANES)], b)
        pj = bufd[i, pl.ds(D, LANES)]
        ci = bufc[i, pl.ds(D, LANES)]
        bufc[i, pl.ds(D, LANES)] = ci - pj
        return 0

    def round_body(r, _):
        blk = r * NW + wid

        @pl.when(blk < NBLK)
        def _():
            e0 = blk * EB
            pltpu.sync_copy(dst_hbm.at[pl.ds(e0, EB)], dstv)
            pltpu.sync_copy(src_hbm.at[pl.ds(e0, EB)], srcv)
            ca = pltpu.async_copy(a1h.at[dstv], bufa, sem0)
            cb = pltpu.async_copy(a2h.at[srcv], bufb, sem1)
            cc = pltpu.async_copy(t1x.at[dstv], bufc, sem2)
            cd = pltpu.async_copy(t2x.at[srcv], bufd, sem3)
            ca.wait()
            cb.wait()
            lax.fori_loop(0, EB, addh_row, 0)
            pltpu.sync_copy(bufa, preh_hbm.at[pl.ds(e0, EB)])
            cc.wait()
            cd.wait()
            lax.fori_loop(0, EB, addx_row, 0)
            pltpu.sync_copy(bufc, prex_hbm.at[pl.ds(e0, EB)])

        return 0

    lax.fori_loop(0, RPW, round_body, 0)


_gather_call = pl.kernel(
    _sc_gather_body,
    out_type=[
        jax.ShapeDtypeStruct((E, D), _f32),
        jax.ShapeDtypeStruct((E, TW), _f32),
    ],
    mesh=plsc.VectorSubcoreMesh(core_axis_name="c", subcore_axis_name="s"),
    scratch_types=[
        pltpu.VMEM((EB,), jnp.int32),
        pltpu.VMEM((EB,), jnp.int32),
        pltpu.VMEM((EB, D), _f32),
        pltpu.VMEM((EB, D), _f32),
        pltpu.VMEM((EB, TW), _f32),
        pltpu.VMEM((EB, TW), _f32),
        pltpu.SemaphoreType.DMA,
        pltpu.SemaphoreType.DMA,
        pltpu.SemaphoreType.DMA,
        pltpu.SemaphoreType.DMA,
    ],
)


NPX = NPAD // 8
ZCX = NPX // NS


def _sc_scatter_body(dst_hbm, msgh_hbm, msgx_hbm, aggh_hbm, aggx_hbm,
                     dstv, dstv8, idxv, idxw, bufm, bufx, shh, shx):
    cid = lax.axis_index("c")
    sid = lax.axis_index("s")

    def fill_idx(base):
        for k in range(EB // LANES):
            idxv[pl.ds(k * LANES, LANES)] = (
                lax.iota(jnp.int32, LANES) + (base + k * LANES))

    def fill_idxw(base):
        for k in range(ZCX // LANES):
            idxw[pl.ds(k * LANES, LANES)] = (
                lax.iota(jnp.int32, LANES) + (base + k * LANES))

    def zero_row(i, _):
        for j in range(D // LANES):
            bufm[i, pl.ds(j * LANES, LANES)] = jnp.zeros((LANES,), _f32)
            bufx[i, pl.ds(j * LANES, LANES)] = jnp.zeros((LANES,), _f32)
        return 0

    lax.fori_loop(0, EB, zero_row, 0)

    r0 = sid * ZCH
    r0x = sid * ZCX
    for c in range(ZCH // EB):
        fill_idx(r0 + c * EB)
        pltpu.sync_copy(bufm, shh.at[idxv])
    fill_idxw(r0x)
    pltpu.sync_copy(bufx.at[pl.ds(0, ZCX)], shx.at[idxw])
    plsc.subcore_barrier()

    def round_body(r, _):
        blk = cid * (NBLK // NC) + r
        e0 = blk * EB
        pltpu.sync_copy(dst_hbm.at[pl.ds(e0, EB)], dstv)
        for k in range(EB // LANES):
            v = dstv[pl.ds(k * LANES, LANES)]
            dstv8[pl.ds(k * LANES, LANES)] = lax.shift_right_logical(v, 3)
        pltpu.sync_copy(msgh_hbm.at[pl.ds(e0, EB)], bufm)
        pltpu.sync_copy(msgx_hbm.at[pl.ds(e0, EB)], bufx)
        pltpu.sync_copy(bufm, shh.at[dstv], add=True)
        pltpu.sync_copy(bufx, shx.at[dstv8], add=True)
        return 0

    @pl.when(sid == 0)
    def _():
        lax.fori_loop(0, NBLK // NC, round_body, 0)

    plsc.subcore_barrier()

    o0 = cid * NPAD + r0
    for c in range(ZCH // EB):
        fill_idx(r0 + c * EB)
        pltpu.sync_copy(shh.at[idxv], bufm)
        pltpu.sync_copy(bufm, aggh_hbm.at[pl.ds(o0 + c * EB, EB)])
    o0x = cid * NPX + r0x
    fill_idxw(r0x)
    pltpu.sync_copy(shx.at[idxw], bufx.at[pl.ds(0, ZCX)])
    pltpu.sync_copy(bufx.at[pl.ds(0, ZCX)], aggx_hbm.at[pl.ds(o0x, ZCX)])


_scatter_call = pl.kernel(
    _sc_scatter_body,
    out_type=[
        jax.ShapeDtypeStruct((NC * NPAD, D), _f32),
        jax.ShapeDtypeStruct((NC * NPX, PW), _f32),
    ],
    mesh=plsc.VectorSubcoreMesh(core_axis_name="c", subcore_axis_name="s"),
    scratch_types=[
        pltpu.VMEM((EB,), jnp.int32),
        pltpu.VMEM((EB,), jnp.int32),
        pltpu.VMEM((EB,), jnp.int32),
        pltpu.VMEM((ZCX,), jnp.int32),
        pltpu.VMEM((EB, D), _f32),
        pltpu.VMEM((EB, PW), _f32),
        pltpu.VMEM_SHARED((NPAD, D), _f32),
        pltpu.VMEM_SHARED((NPX, PW), _f32),
    ],
)


def _tables_body(h, pos128, w1hi, w1hj, b1h, w1xi, w1xj, b1x,
                 a1h, a2h, t1x, t2x):
    hv = h[...]
    pv = pos128[...]
    a1h[...] = _dot(hv, w1hi[...]) + b1h[...]
    a2h[...] = _dot(hv, w1hj[...])
    a1xv = _dot(hv, w1xi[...]) + b1x[...]
    a2xv = _dot(hv, w1xj[...])
    t1x[...] = jnp.concatenate([a1xv, pv], axis=1)
    t2x[...] = jnp.concatenate([a2xv, pv], axis=1)


def _wspec():
    return pl.BlockSpec((D, D), lambda i: (0, 0))


def _bspec():
    return pl.BlockSpec((1, D), lambda i: (0, 0))


_tables_call = pl.pallas_call(
    _tables_body,
    grid=(N // BN,),
    in_specs=[
        pl.BlockSpec((BN, D), lambda i: (i, 0)),
        pl.BlockSpec((BN, PW), lambda i: (i, 0)),
        _wspec(), _wspec(), _bspec(), _wspec(), _wspec(), _bspec(),
    ],
    out_specs=[
        pl.BlockSpec((BN, D), lambda i: (i, 0)),
        pl.BlockSpec((BN, D), lambda i: (i, 0)),
        pl.BlockSpec((BN, TW), lambda i: (i, 0)),
        pl.BlockSpec((BN, TW), lambda i: (i, 0)),
    ],
    out_shape=[
        jax.ShapeDtypeStruct((N, D), _f32),
        jax.ShapeDtypeStruct((N, D), _f32),
        jax.ShapeDtypeStruct((N, TW), _f32),
        jax.ShapeDtypeStruct((N, TW), _f32),
    ],
)


def _edge_body(preh, prex, dst3, w2h, b2h, wa, ba, w2x, b2x, w3x, b3x,
               wdh, wdx, msgh, msgx):
    prexv = prex[...]
    dxyz = prexv[:, D:D + XW]
    d2 = jnp.sum(dxyz * dxyz, axis=1, keepdims=True)
    m1 = _silu(preh[...] + d2 * wdh[...])
    m2 = _silu(_dot(m1, w2h[...]) + b2h[...])
    attn = jax.nn.sigmoid(
        _dot(m2, wa[...]) + ba[...])
    msgh[...] = attn * m2
    t1 = _silu(prexv[:, :D] + d2 * wdx[...])
    t2 = _silu(_dot(t1, w2x[...]) + b2x[...])
    s = _dot(t2, w3x[...]) + b3x[...]
    mx = dxyz * (s / (jnp.sqrt(d2) + 1.0))
    m8 = (dst3[...].reshape(BE, 1)) % 8
    laneblk = lax.broadcasted_iota(jnp.int32, (BE, PW), 1) // XW
    tiled8 = jnp.concatenate([mx] * (PW // XW), axis=1)
    msgx[...] = jnp.where(laneblk == m8, tiled8, 0.0)


_edge_call = pl.pallas_call(
    _edge_body,
    grid=(E // BE,),
    in_specs=[
        pl.BlockSpec((BE, D), lambda i: (i, 0)),
        pl.BlockSpec((BE, TW), lambda i: (i, 0)),
        pl.BlockSpec((1, 1, BE), lambda i: (i, 0, 0)),
        _wspec(), _bspec(),
        pl.BlockSpec((D, 1), lambda i: (0, 0)),
        pl.BlockSpec((1, 1), lambda i: (0, 0)),
        _wspec(), _bspec(),
        pl.BlockSpec((D, 1), lambda i: (0, 0)),
        pl.BlockSpec((1, 1), lambda i: (0, 0)),
        _bspec(), _bspec(),
    ],
    out_specs=[
        pl.BlockSpec((BE, D), lambda i: (i, 0)),
        pl.BlockSpec((BE, PW), lambda i: (i, 0)),
    ],
    out_shape=[
        jax.ShapeDtypeStruct((E, D), _f32),
        jax.ShapeDtypeStruct((E, PW), _f32),
    ],
)


def _node_body(h, agghp, aggxp, pos128, wu1a, wu1b, bu1, wu2, bu2,
               w1hi, w1hj, b1h, w1xi, w1xj, b1x,
               hn, pos128n, a1h, a2h, t1x, t2x):
    hv = h[...]
    ap = agghp[...]
    aggh = ap[0] + ap[1]
    u = _silu(_dot(hv, wu1a[...])
              + _dot(aggh, wu1b[...])
              + bu1[...])
    hnv = hv + _dot(u, wu2[...]) + bu2[...]
    hn[...] = hnv
    xp = aggxp[...]
    dx = jnp.pad(xp[0] + xp[1], ((0, 0), (0, PW - XW)))
    pnv = pos128[...] + dx
    pos128n[...] = pnv
    a1h[...] = _dot(hnv, w1hi[...]) + b1h[...]
    a2h[...] = _dot(hnv, w1hj[...])
    a1xv = _dot(hnv, w1xi[...]) + b1x[...]
    a2xv = _dot(hnv, w1xj[...])
    t1x[...] = jnp.concatenate([a1xv, pnv], axis=1)
    t2x[...] = jnp.concatenate([a2xv, pnv], axis=1)


_node_call = pl.pallas_call(
    _node_body,
    grid=(N // BN,),
    in_specs=[
        pl.BlockSpec((BN, D), lambda i: (i, 0)),
        pl.BlockSpec((NC, BN, D), lambda i: (0, i, 0)),
        pl.BlockSpec((NC, BN, XW), lambda i: (0, i, 0)),
        pl.BlockSpec((BN, PW), lambda i: (i, 0)),
        _wspec(), _wspec(), _bspec(), _wspec(), _bspec(),
        _wspec(), _wspec(), _bspec(), _wspec(), _wspec(), _bspec(),
    ],
    out_specs=[
        pl.BlockSpec((BN, D), lambda i: (i, 0)),
        pl.BlockSpec((BN, PW), lambda i: (i, 0)),
        pl.BlockSpec((BN, D), lambda i: (i, 0)),
        pl.BlockSpec((BN, D), lambda i: (i, 0)),
        pl.BlockSpec((BN, TW), lambda i: (i, 0)),
        pl.BlockSpec((BN, TW), lambda i: (i, 0)),
    ],
    out_shape=[
        jax.ShapeDtypeStruct((N, D), _f32),
        jax.ShapeDtypeStruct((N, PW), _f32),
        jax.ShapeDtypeStruct((N, D), _f32),
        jax.ShapeDtypeStruct((N, D), _f32),
        jax.ShapeDtypeStruct((N, TW), _f32),
        jax.ShapeDtypeStruct((N, TW), _f32),
    ],
)


def _final_body(h, agghp, batch, wu1a, wu1b, bu1, wu2, bu2, wpredt, bpred,
                sums, counts, out):
    i = pl.program_id(0)
    hv = h[...]
    ap = agghp[...]
    aggh = ap[0] + ap[1]
    u = _silu(_dot(hv, wu1a[...])
              + _dot(aggh, wu1b[...])
              + bu1[...])
    hnv = hv + _dot(u, wu2[...]) + bu2[...]
    b = batch[...].reshape(1, BN)
    oh = (lax.broadcasted_iota(jnp.int32, (G, BN), 0) == b).astype(_f32)
    psum = _dot(oh, hnv)
    pcnt = jnp.sum(oh, axis=1, keepdims=True)

    @pl.when(i == 0)
    def _():
        sums[...] = jnp.zeros_like(sums)
        counts[...] = jnp.zeros_like(counts)

    sums[...] += psum
    counts[...] += pcnt

    @pl.when(i == pl.num_programs(0) - 1)
    def _():
        hg = sums[...] / jnp.maximum(counts[...], 1.0)
        out[...] = _dot(hg, wpredt[...]) + bpred[...]


_final_call = pl.pallas_call(
    _final_body,
    grid=(N // BN,),
    in_specs=[
        pl.BlockSpec((BN, D), lambda i: (i, 0)),
        pl.BlockSpec((NC, BN, D), lambda i: (0, i, 0)),
        pl.BlockSpec((1, 1, BN), lambda i: (i, 0, 0)),
        _wspec(), _wspec(), _bspec(), _wspec(), _bspec(),
        pl.BlockSpec((D, 1), lambda i: (0, 0)),
        pl.BlockSpec((1, 1), lambda i: (0, 0)),
    ],
    out_specs=[
        pl.BlockSpec((G, D), lambda i: (0, 0)),
        pl.BlockSpec((G, 1), lambda i: (0, 0)),
        pl.BlockSpec((G, 1), lambda i: (0, 0)),
    ],
    out_shape=[
        jax.ShapeDtypeStruct((G, D), _f32),
        jax.ShapeDtypeStruct((G, 1), _f32),
        jax.ShapeDtypeStruct((G, 1), _f32),
    ],
)


def kernel(x, pos, edge_index, batch,
           W_msg_h1, b_msg_h1, W_msg_h2, b_msg_h2, W_attn, b_attn,
           W_upd1, b_upd1, W_upd2, b_upd2,
           W_msgx1, b_msgx1, W_msgx2, b_msgx2, W_msgx3, b_msgx3,
           W_pred, b_pred):
    src = edge_index[0]
    dst = edge_index[1]
    dst3 = dst.reshape(E // BE, 1, BE)
    h = x
    pos128 = jnp.pad(pos, ((0, 0), (0, PW - 3)))

    def layer_w(l):
        w1h = W_msg_h1[l]
        w1x = W_msgx1[l]
        return dict(
            w1hi=w1h[:, :D].T, w1hj=w1h[:, D:2 * D].T,
            wdh=w1h[:, 2 * D:].T, b1h=b_msg_h1[l].reshape(1, D),
            w1xi=w1x[:, :D].T, w1xj=w1x[:, D:2 * D].T,
            wdx=w1x[:, 2 * D:].T, b1x=b_msgx1[l].reshape(1, D),
            w2h=W_msg_h2[l].T, b2h=b_msg_h2[l].reshape(1, D),
            wa=W_attn[l].T, ba=b_attn[l].reshape(1, 1),
            w2x=W_msgx2[l].T, b2x=b_msgx2[l].reshape(1, D),
            w3x=W_msgx3[l].T, b3x=b_msgx3[l].reshape(1, 1),
            wu1a=W_upd1[l][:, :D].T, wu1b=W_upd1[l][:, D:].T,
            bu1=b_upd1[l].reshape(1, D),
            wu2=W_upd2[l].T, bu2=b_upd2[l].reshape(1, D),
        )

    ws = [layer_w(l) for l in range(NLAYERS)]
    w0 = ws[0]
    a1h, a2h, t1x, t2x = _tables_call(
        h, pos128, w0["w1hi"], w0["w1hj"], w0["b1h"],
        w0["w1xi"], w0["w1xj"], w0["b1x"])

    out = None
    for l in range(NLAYERS):
        w = ws[l]
        preh, prex = _gather_call(dst, src, a1h, a2h, t1x, t2x)
        msgh, msgx = _edge_call(
            preh, prex, dst3, w["w2h"], w["b2h"], w["wa"], w["ba"],
            w["w2x"], w["b2x"], w["w3x"], w["b3x"], w["wdh"], w["wdx"])
        if _DBG_XLA_SCATTER:
            _aggh = jax.ops.segment_sum(msgh, dst, num_segments=N)
            _aggx = jax.ops.segment_sum(msgx, dst, num_segments=N)
            agghp = jnp.stack([jnp.pad(_aggh, ((0, NPAD - N), (0, 0))),
                               jnp.zeros((NPAD, D), _f32)])
            aggxp = jnp.stack([jnp.pad(_aggx, ((0, NPAD - N), (0, 0))),
                               jnp.zeros((NPAD, XW), _f32)])
        else:
            agghp, aggxp = _scatter_call(dst, msgh, msgx)
            agghp = agghp.reshape(NC, NPAD, D)
            aggxp = aggxp.reshape(NC, NPAD, XW)
        if l < NLAYERS - 1:
            wn = ws[l + 1]
            h, pos128, a1h, a2h, t1x, t2x = _node_call(
                h, agghp, aggxp, pos128,
                w["wu1a"], w["wu1b"], w["bu1"], w["wu2"], w["bu2"],
                wn["w1hi"], wn["w1hj"], wn["b1h"],
                wn["w1xi"], wn["w1xj"], wn["b1x"])
        else:
            batch3 = batch.reshape(N // BN, 1, BN)
            _sums, _counts, out = _final_call(
                h, agghp, batch3,
                w["wu1a"], w["wu1b"], w["bu1"], w["wu2"], w["bu2"],
                W_pred.T, b_pred.reshape(1, 1))
    return out.reshape(-1)


# speedup vs baseline: 1.0031x; 1.0031x over previous
import jax
import jax.numpy as jnp
from jax import lax
from jax.experimental import pallas as pl
from jax.experimental.pallas import tpu as pltpu
from jax.experimental.pallas import tpu_sc as plsc

N = 10000
E = 320000
D = 128
NLAYERS = 4
G = 16

NC, NS, LANES = 2, 16, 16
NW = NC * NS
EB = 128
NBLK = E // EB
RPW = (NBLK + NW - 1) // NW
ZCH = 640
NPAD = NS * ZCH

BN = 2000
BE = 2560
XW = 16
PW = 128
TW = D + PW

_f32 = jnp.float32
_DBG_XLA_SCATTER = False


def _silu(v):
    return v * jax.nn.sigmoid(v)


def _dot(a, b):
    return jnp.dot(a, b, preferred_element_type=_f32,
                   precision=jax.lax.Precision.HIGHEST)


def _sc_gather_body(dst_hbm, src_hbm, a1h, a2h, t1x, t2x,
                    preh_hbm, prex_hbm,
                    dstv, srcv, bufa, bufb, bufc, bufd,
                    sem0, sem1, sem2, sem3):
    cid = lax.axis_index("c")
    sid = lax.axis_index("s")
    wid = sid * NC + cid

    def addh_row(i, _):
        for j in range(D // LANES):
            b = bufb[i, pl.ds(j * LANES, LANES)]
            plsc.addupdate(bufa.at[i, pl.ds(j * LANES, LANES)], b)
        return 0

    def addx_row(i, _):
        for j in range(D // LANES):
            b = bufd[i, pl.ds(j * LANES, LANES)]
            plsc.addupdate(bufc.at[i, pl.ds(j * LANES, LANES)], b)
        pj = bufd[i, pl.ds(D, LANES)]
        ci = bufc[i, pl.ds(D, LANES)]
        bufc[i, pl.ds(D, LANES)] = ci - pj
        return 0

    def round_body(r, _):
        blk = r * NW + wid

        @pl.when(blk < NBLK)
        def _():
            e0 = blk * EB
            pltpu.sync_copy(dst_hbm.at[pl.ds(e0, EB)], dstv)
            pltpu.sync_copy(src_hbm.at[pl.ds(e0, EB)], srcv)
            ca = pltpu.async_copy(a1h.at[dstv], bufa, sem0)
            cb = pltpu.async_copy(a2h.at[srcv], bufb, sem1)
            cc = pltpu.async_copy(t1x.at[dstv], bufc, sem2)
            cd = pltpu.async_copy(t2x.at[srcv], bufd, sem3)
            ca.wait()
            cb.wait()
            lax.fori_loop(0, EB, addh_row, 0)
            pltpu.sync_copy(bufa, preh_hbm.at[pl.ds(e0, EB)])
            cc.wait()
            cd.wait()
            lax.fori_loop(0, EB, addx_row, 0)
            pltpu.sync_copy(bufc, prex_hbm.at[pl.ds(e0, EB)])

        return 0

    lax.fori_loop(0, RPW, round_body, 0)


_gather_call = pl.kernel(
    _sc_gather_body,
    out_type=[
        jax.ShapeDtypeStruct((E, D), _f32),
        jax.ShapeDtypeStruct((E, TW), _f32),
    ],
    mesh=plsc.VectorSubcoreMesh(core_axis_name="c", subcore_axis_name="s"),
    scratch_types=[
        pltpu.VMEM((EB,), jnp.int32),
        pltpu.VMEM((EB,), jnp.int32),
        pltpu.VMEM((EB, D), _f32),
        pltpu.VMEM((EB, D), _f32),
        pltpu.VMEM((EB, TW), _f32),
        pltpu.VMEM((EB, TW), _f32),
        pltpu.SemaphoreType.DMA,
        pltpu.SemaphoreType.DMA,
        pltpu.SemaphoreType.DMA,
        pltpu.SemaphoreType.DMA,
    ],
)


NPX = NPAD // 8
ZCX = NPX // NS


def _sc_scatter_body(dst_hbm, msgh_hbm, msgx_hbm, aggh_hbm, aggx_hbm,
                     dstv0, dstv1, idxv, idxw, bufm0, bufm1,
                     shh, shx,
                     smd0, smm0, sma0, smd1, smm1, sma1):
    cid = lax.axis_index("c")
    sid = lax.axis_index("s")
    NBC = NBLK // NC

    def fill_idx(base):
        for k in range(EB // LANES):
            idxv[pl.ds(k * LANES, LANES)] = (
                lax.iota(jnp.int32, LANES) + (base + k * LANES))

    def fill_idxw(base):
        for k in range(ZCX // LANES):
            idxw[pl.ds(k * LANES, LANES)] = (
                lax.iota(jnp.int32, LANES) + (base + k * LANES))

    def zero_row(i, _):
        for j in range(D // LANES):
            bufm0[i, pl.ds(j * LANES, LANES)] = jnp.zeros((LANES,), _f32)
        return 0

    lax.fori_loop(0, EB, zero_row, 0)

    r0 = sid * ZCH
    r0x = sid * ZCX
    for c in range(ZCH // EB):
        fill_idx(r0 + c * EB)
        pltpu.sync_copy(bufm0, shh.at[idxv])
    fill_idxw(r0x)
    pltpu.sync_copy(bufm0.at[pl.ds(0, ZCX)], shx.at[idxw])
    plsc.subcore_barrier()

    NBC = NBLK // NC

    def round_body(r, _):
        e0 = (cid * NBC + r) * EB
        pltpu.sync_copy(dst_hbm.at[pl.ds(e0, EB)], dstv0)
        for k in range(EB // LANES):
            v = dstv0[pl.ds(k * LANES, LANES)]
            dstv1[pl.ds(k * LANES, LANES)] = lax.shift_right_logical(v, 3)
        pltpu.sync_copy(msgh_hbm.at[pl.ds(e0, EB)], bufm0)
        pltpu.sync_copy(msgx_hbm.at[pl.ds(e0, EB)], bufm1)
        pltpu.sync_copy(bufm0, shh.at[dstv0], add=True)
        pltpu.sync_copy(bufm1, shx.at[dstv1], add=True)
        return 0

    @pl.when(sid == 0)
    def _():
        lax.fori_loop(0, NBC, round_body, 0)

    plsc.subcore_barrier()

    o0 = cid * NPAD + r0
    for c in range(ZCH // EB):
        fill_idx(r0 + c * EB)
        pltpu.sync_copy(shh.at[idxv], bufm0)
        pltpu.sync_copy(bufm0, aggh_hbm.at[pl.ds(o0 + c * EB, EB)])
    o0x = cid * NPX + r0x
    fill_idxw(r0x)
    pltpu.sync_copy(shx.at[idxw], bufm0.at[pl.ds(0, ZCX)])
    pltpu.sync_copy(bufm0.at[pl.ds(0, ZCX)], aggx_hbm.at[pl.ds(o0x, ZCX)])


_scatter_call = pl.kernel(
    _sc_scatter_body,
    out_type=[
        jax.ShapeDtypeStruct((NC * NPAD, D), _f32),
        jax.ShapeDtypeStruct((NC * NPX, PW), _f32),
    ],
    mesh=plsc.VectorSubcoreMesh(core_axis_name="c", subcore_axis_name="s"),
    scratch_types=[
        pltpu.VMEM((EB,), jnp.int32),
        pltpu.VMEM((EB,), jnp.int32),
        pltpu.VMEM((EB,), jnp.int32),
        pltpu.VMEM((ZCX,), jnp.int32),
        pltpu.VMEM((EB, D), _f32),
        pltpu.VMEM((EB, D), _f32),
        pltpu.VMEM_SHARED((NPAD, D), _f32),
        pltpu.VMEM_SHARED((NPX, PW), _f32),
        pltpu.SemaphoreType.DMA,
        pltpu.SemaphoreType.DMA,
        pltpu.SemaphoreType.DMA,
        pltpu.SemaphoreType.DMA,
        pltpu.SemaphoreType.DMA,
        pltpu.SemaphoreType.DMA,
    ],
)


def _tables_body(h, pos128, w1hi, w1hj, b1h, w1xi, w1xj, b1x,
                 a1h, a2h, t1x, t2x):
    hv = h[...]
    pv = pos128[...]
    a1h[...] = _dot(hv, w1hi[...]) + b1h[...]
    a2h[...] = _dot(hv, w1hj[...])
    a1xv = _dot(hv, w1xi[...]) + b1x[...]
    a2xv = _dot(hv, w1xj[...])
    t1x[...] = jnp.concatenate([a1xv, pv], axis=1)
    t2x[...] = jnp.concatenate([a2xv, pv], axis=1)


def _wspec():
    return pl.BlockSpec((D, D), lambda i: (0, 0))


def _bspec():
    return pl.BlockSpec((1, D), lambda i: (0, 0))


_tables_call = pl.pallas_call(
    _tables_body,
    grid=(N // BN,),
    in_specs=[
        pl.BlockSpec((BN, D), lambda i: (i, 0)),
        pl.BlockSpec((BN, PW), lambda i: (i, 0)),
        _wspec(), _wspec(), _bspec(), _wspec(), _wspec(), _bspec(),
    ],
    out_specs=[
        pl.BlockSpec((BN, D), lambda i: (i, 0)),
        pl.BlockSpec((BN, D), lambda i: (i, 0)),
        pl.BlockSpec((BN, TW), lambda i: (i, 0)),
        pl.BlockSpec((BN, TW), lambda i: (i, 0)),
    ],
    out_shape=[
        jax.ShapeDtypeStruct((N, D), _f32),
        jax.ShapeDtypeStruct((N, D), _f32),
        jax.ShapeDtypeStruct((N, TW), _f32),
        jax.ShapeDtypeStruct((N, TW), _f32),
    ],
)


def _edge_body(preh, prex, dst3, w2h, b2h, wa, ba, w2x, b2x, w3x, b3x,
               wdh, wdx, msgh, msgx):
    prexv = prex[...]
    dxyz = prexv[:, D:D + XW]
    d2 = jnp.sum(dxyz * dxyz, axis=1, keepdims=True)
    m1 = _silu(preh[...] + d2 * wdh[...])
    m2 = _silu(_dot(m1, w2h[...]) + b2h[...])
    attn = jax.nn.sigmoid(
        _dot(m2, wa[...]) + ba[...])
    msgh[...] = attn * m2
    t1 = _silu(prexv[:, :D] + d2 * wdx[...])
    t2 = _silu(_dot(t1, w2x[...]) + b2x[...])
    s = _dot(t2, w3x[...]) + b3x[...]
    mx = dxyz * (s / (jnp.sqrt(d2) + 1.0))
    m8 = (dst3[...].reshape(BE, 1)) % 8
    laneblk = lax.broadcasted_iota(jnp.int32, (BE, PW), 1) // XW
    tiled8 = jnp.concatenate([mx] * (PW // XW), axis=1)
    msgx[...] = jnp.where(laneblk == m8, tiled8, 0.0)


_edge_call = pl.pallas_call(
    _edge_body,
    grid=(E // BE,),
    in_specs=[
        pl.BlockSpec((BE, D), lambda i: (i, 0)),
        pl.BlockSpec((BE, TW), lambda i: (i, 0)),
        pl.BlockSpec((1, 1, BE), lambda i: (i, 0, 0)),
        _wspec(), _bspec(),
        pl.BlockSpec((D, 1), lambda i: (0, 0)),
        pl.BlockSpec((1, 1), lambda i: (0, 0)),
        _wspec(), _bspec(),
        pl.BlockSpec((D, 1), lambda i: (0, 0)),
        pl.BlockSpec((1, 1), lambda i: (0, 0)),
        _bspec(), _bspec(),
    ],
    out_specs=[
        pl.BlockSpec((BE, D), lambda i: (i, 0)),
        pl.BlockSpec((BE, PW), lambda i: (i, 0)),
    ],
    out_shape=[
        jax.ShapeDtypeStruct((E, D), _f32),
        jax.ShapeDtypeStruct((E, PW), _f32),
    ],
)


def _node_body(h, agghp, aggxp, pos128, wu1a, wu1b, bu1, wu2, bu2,
               w1hi, w1hj, b1h, w1xi, w1xj, b1x,
               hn, pos128n, a1h, a2h, t1x, t2x):
    hv = h[...]
    ap = agghp[...]
    aggh = ap[0] + ap[1]
    u = _silu(_dot(hv, wu1a[...])
              + _dot(aggh, wu1b[...])
              + bu1[...])
    hnv = hv + _dot(u, wu2[...]) + bu2[...]
    hn[...] = hnv
    xp = aggxp[...]
    dx = jnp.pad(xp[0] + xp[1], ((0, 0), (0, PW - XW)))
    pnv = pos128[...] + dx
    pos128n[...] = pnv
    a1h[...] = _dot(hnv, w1hi[...]) + b1h[...]
    a2h[...] = _dot(hnv, w1hj[...])
    a1xv = _dot(hnv, w1xi[...]) + b1x[...]
    a2xv = _dot(hnv, w1xj[...])
    t1x[...] = jnp.concatenate([a1xv, pnv], axis=1)
    t2x[...] = jnp.concatenate([a2xv, pnv], axis=1)


_node_call = pl.pallas_call(
    _node_body,
    grid=(N // BN,),
    in_specs=[
        pl.BlockSpec((BN, D), lambda i: (i, 0)),
        pl.BlockSpec((NC, BN, D), lambda i: (0, i, 0)),
        pl.BlockSpec((NC, BN, XW), lambda i: (0, i, 0)),
        pl.BlockSpec((BN, PW), lambda i: (i, 0)),
        _wspec(), _wspec(), _bspec(), _wspec(), _bspec(),
        _wspec(), _wspec(), _bspec(), _wspec(), _wspec(), _bspec(),
    ],
    out_specs=[
        pl.BlockSpec((BN, D), lambda i: (i, 0)),
        pl.BlockSpec((BN, PW), lambda i: (i, 0)),
        pl.BlockSpec((BN, D), lambda i: (i, 0)),
        pl.BlockSpec((BN, D), lambda i: (i, 0)),
        pl.BlockSpec((BN, TW), lambda i: (i, 0)),
        pl.BlockSpec((BN, TW), lambda i: (i, 0)),
    ],
    out_shape=[
        jax.ShapeDtypeStruct((N, D), _f32),
        jax.ShapeDtypeStruct((N, PW), _f32),
        jax.ShapeDtypeStruct((N, D), _f32),
        jax.ShapeDtypeStruct((N, D), _f32),
        jax.ShapeDtypeStruct((N, TW), _f32),
        jax.ShapeDtypeStruct((N, TW), _f32),
    ],
)


def _final_body(h, agghp, batch, wu1a, wu1b, bu1, wu2, bu2, wpredt, bpred,
                sums, counts, out):
    i = pl.program_id(0)
    hv = h[...]
    ap = agghp[...]
    aggh = ap[0] + ap[1]
    u = _silu(_dot(hv, wu1a[...])
              + _dot(aggh, wu1b[...])
              + bu1[...])
    hnv = hv + _dot(u, wu2[...]) + bu2[...]
    b = batch[...].reshape(1, BN)
    oh = (lax.broadcasted_iota(jnp.int32, (G, BN), 0) == b).astype(_f32)
    psum = _dot(oh, hnv)
    pcnt = jnp.sum(oh, axis=1, keepdims=True)

    @pl.when(i == 0)
    def _():
        sums[...] = jnp.zeros_like(sums)
        counts[...] = jnp.zeros_like(counts)

    sums[...] += psum
    counts[...] += pcnt

    @pl.when(i == pl.num_programs(0) - 1)
    def _():
        hg = sums[...] / jnp.maximum(counts[...], 1.0)
        out[...] = _dot(hg, wpredt[...]) + bpred[...]


_final_call = pl.pallas_call(
    _final_body,
    grid=(N // BN,),
    in_specs=[
        pl.BlockSpec((BN, D), lambda i: (i, 0)),
        pl.BlockSpec((NC, BN, D), lambda i: (0, i, 0)),
        pl.BlockSpec((1, 1, BN), lambda i: (i, 0, 0)),
        _wspec(), _wspec(), _bspec(), _wspec(), _bspec(),
        pl.BlockSpec((D, 1), lambda i: (0, 0)),
        pl.BlockSpec((1, 1), lambda i: (0, 0)),
    ],
    out_specs=[
        pl.BlockSpec((G, D), lambda i: (0, 0)),
        pl.BlockSpec((G, 1), lambda i: (0, 0)),
        pl.BlockSpec((G, 1), lambda i: (0, 0)),
    ],
    out_shape=[
        jax.ShapeDtypeStruct((G, D), _f32),
        jax.ShapeDtypeStruct((G, 1), _f32),
        jax.ShapeDtypeStruct((G, 1), _f32),
    ],
)


def kernel(x, pos, edge_index, batch,
           W_msg_h1, b_msg_h1, W_msg_h2, b_msg_h2, W_attn, b_attn,
           W_upd1, b_upd1, W_upd2, b_upd2,
           W_msgx1, b_msgx1, W_msgx2, b_msgx2, W_msgx3, b_msgx3,
           W_pred, b_pred):
    src = edge_index[0]
    dst = edge_index[1]
    dst3 = dst.reshape(E // BE, 1, BE)
    h = x
    pos128 = jnp.pad(pos, ((0, 0), (0, PW - 3)))

    def layer_w(l):
        w1h = W_msg_h1[l]
        w1x = W_msgx1[l]
        return dict(
            w1hi=w1h[:, :D].T, w1hj=w1h[:, D:2 * D].T,
            wdh=w1h[:, 2 * D:].T, b1h=b_msg_h1[l].reshape(1, D),
            w1xi=w1x[:, :D].T, w1xj=w1x[:, D:2 * D].T,
            wdx=w1x[:, 2 * D:].T, b1x=b_msgx1[l].reshape(1, D),
            w2h=W_msg_h2[l].T, b2h=b_msg_h2[l].reshape(1, D),
            wa=W_attn[l].T, ba=b_attn[l].reshape(1, 1),
            w2x=W_msgx2[l].T, b2x=b_msgx2[l].reshape(1, D),
            w3x=W_msgx3[l].T, b3x=b_msgx3[l].reshape(1, 1),
            wu1a=W_upd1[l][:, :D].T, wu1b=W_upd1[l][:, D:].T,
            bu1=b_upd1[l].reshape(1, D),
            wu2=W_upd2[l].T, bu2=b_upd2[l].reshape(1, D),
        )

    ws = [layer_w(l) for l in range(NLAYERS)]
    w0 = ws[0]
    a1h, a2h, t1x, t2x = _tables_call(
        h, pos128, w0["w1hi"], w0["w1hj"], w0["b1h"],
        w0["w1xi"], w0["w1xj"], w0["b1x"])

    out = None
    for l in range(NLAYERS):
        w = ws[l]
        preh, prex = _gather_call(dst, src, a1h, a2h, t1x, t2x)
        msgh, msgx = _edge_call(
            preh, prex, dst3, w["w2h"], w["b2h"], w["wa"], w["ba"],
            w["w2x"], w["b2x"], w["w3x"], w["b3x"], w["wdh"], w["wdx"])
        if _DBG_XLA_SCATTER:
            _aggh = jax.ops.segment_sum(msgh, dst, num_segments=N)
            _aggx = jax.ops.segment_sum(msgx, dst, num_segments=N)
            agghp = jnp.stack([jnp.pad(_aggh, ((0, NPAD - N), (0, 0))),
                               jnp.zeros((NPAD, D), _f32)])
            aggxp = jnp.stack([jnp.pad(_aggx, ((0, NPAD - N), (0, 0))),
                               jnp.zeros((NPAD, XW), _f32)])
        else:
            agghp, aggxp = _scatter_call(dst, msgh, msgx)
            agghp = agghp.reshape(NC, NPAD, D)
            aggxp = aggxp.reshape(NC, NPAD, XW)
        if l < NLAYERS - 1:
            wn = ws[l + 1]
            h, pos128, a1h, a2h, t1x, t2x = _node_call(
                h, agghp, aggxp, pos128,
                w["wu1a"], w["wu1b"], w["bu1"], w["wu2"], w["bu2"],
                wn["w1hi"], wn["w1hj"], wn["b1h"],
                wn["w1xi"], wn["w1xj"], wn["b1x"])
        else:
            batch3 = batch.reshape(N // BN, 1, BN)
            _sums, _counts, out = _final_call(
                h, agghp, batch3,
                w["wu1a"], w["wu1b"], w["bu1"], w["wu2"], w["bu2"],
                W_pred.T, b_pred.reshape(1, 1))
    return out.reshape(-1)
